# Initial kernel scaffold; baseline (speedup 1.0000x reference)
import functools

import jax
import jax.numpy as jnp
from jax import lax
from jax.experimental import pallas as pl
from jax.experimental.pallas import tpu as pltpu
from jax.experimental.pallas import tpu_sc as plsc

EMBED_DIM = 32
CHUNK = 128
K = 8


def _sc_gather(table, idx2d, n_chunks, n_groups, NC, NS):
    NW = NC * NS
    B = NW * n_chunks * CHUNK
    mesh = plsc.VectorSubcoreMesh(core_axis_name="c", subcore_axis_name="s")

    @functools.partial(
        pl.kernel,
        mesh=mesh,
        out_type=jax.ShapeDtypeStruct((B, EMBED_DIM), jnp.float32),
        scratch_types=[
            pltpu.VMEM((n_chunks, CHUNK), jnp.int32),
            pltpu.VMEM((K * CHUNK, EMBED_DIM), jnp.float32),
            pltpu.SemaphoreType.DMA,
        ],
    )
    def body(table_hbm, idx_hbm, out_hbm, idx_v, rows_v, sem):
        wid = lax.axis_index("s") * NC + lax.axis_index("c")
        base_chunk = wid * n_chunks
        pltpu.sync_copy(idx_hbm.at[pl.ds(base_chunk, n_chunks)], idx_v)

        def group(g, carry):
            copies = []
            for j in range(K):
                copies.append(
                    pltpu.async_copy(
                        table_hbm.at[idx_v.at[g * K + j]],
                        rows_v.at[pl.ds(j * CHUNK, CHUNK)],
                        sem,
                    )
                )
            for c in copies:
                c.wait()
            out_base = (base_chunk + g * K) * CHUNK
            pltpu.sync_copy(rows_v, out_hbm.at[pl.ds(out_base, K * CHUNK)])
            return carry

        lax.fori_loop(0, n_groups, group, 0)

    return body(table, idx2d)


def kernel(x, table):
    B, H = x.shape
    total = B * H
    info = plsc.get_sparse_core_info()
    NC, NS = info.num_cores, info.num_subcores
    NW = NC * NS
    n_chunks = total // (NW * CHUNK)
    n_groups = n_chunks // K
    idx_flat = x.reshape(total).astype(jnp.int32)
    idx2d = idx_flat.reshape(NW * n_chunks, CHUNK)
    out = _sc_gather(table, idx2d, n_chunks, n_groups, NC, NS)
    return out.reshape(B, H, EMBED_DIM)


# speedup vs baseline: 1.1013x; 1.1013x over previous
import functools

import jax
import jax.numpy as jnp
from jax import lax
from jax.experimental import pallas as pl
from jax.experimental.pallas import tpu as pltpu
from jax.experimental.pallas import tpu_sc as plsc

EMBED_DIM = 32
CHUNK = 128
K = 8


def _sc_gather(table, idx2d, n_chunks, n_groups, NC, NS):
    NW = NC * NS
    B = NW * n_chunks * CHUNK
    mesh = plsc.VectorSubcoreMesh(core_axis_name="c", subcore_axis_name="s")

    @functools.partial(
        pl.kernel,
        mesh=mesh,
        out_type=jax.ShapeDtypeStruct((B, EMBED_DIM), jnp.float32),
        scratch_types=[
            pltpu.VMEM((n_chunks, CHUNK), jnp.int32),
            pltpu.VMEM((K * CHUNK, EMBED_DIM), jnp.float32),
            pltpu.SemaphoreType.DMA,
        ],
        compiler_params=pltpu.CompilerParams(use_tc_tiling_on_sc=False),
    )
    def body(table_hbm, idx_hbm, out_hbm, idx_v, rows_v, sem):
        wid = lax.axis_index("s") * NC + lax.axis_index("c")
        base_chunk = wid * n_chunks
        pltpu.sync_copy(idx_hbm.at[pl.ds(base_chunk, n_chunks)], idx_v)

        def group(g, carry):
            copies = []
            for j in range(K):
                copies.append(
                    pltpu.async_copy(
                        table_hbm.at[idx_v.at[g * K + j]],
                        rows_v.at[pl.ds(j * CHUNK, CHUNK)],
                        sem,
                    )
                )
            for c in copies:
                c.wait()
            out_base = (base_chunk + g * K) * CHUNK
            pltpu.sync_copy(rows_v, out_hbm.at[pl.ds(out_base, K * CHUNK)])
            return carry

        lax.fori_loop(0, n_groups, group, 0)

    return body(table, idx2d)


def kernel(x, table):
    B, H = x.shape
    total = B * H
    info = plsc.get_sparse_core_info()
    NC, NS = info.num_cores, info.num_subcores
    NW = NC * NS
    n_chunks = total // (NW * CHUNK)
    n_groups = n_chunks // K
    idx_flat = x.reshape(total).astype(jnp.int32)
    idx2d = idx_flat.reshape(NW * n_chunks, CHUNK)
    out = _sc_gather(table, idx2d, n_chunks, n_groups, NC, NS)
    return out.reshape(B, H, EMBED_DIM)


# speedup vs baseline: 1.1115x; 1.0093x over previous
import functools

import jax
import jax.numpy as jnp
from jax import lax
from jax.experimental import pallas as pl
from jax.experimental.pallas import tpu as pltpu
from jax.experimental.pallas import tpu_sc as plsc

EMBED_DIM = 32
CHUNK = 128
K = 4


def _sc_gather(table, idx2d, n_chunks, n_groups, NC, NS):
    NW = NC * NS
    B = NW * n_chunks * CHUNK
    GROUP = K * CHUNK
    mesh = plsc.VectorSubcoreMesh(core_axis_name="c", subcore_axis_name="s")

    @functools.partial(
        pl.kernel,
        mesh=mesh,
        out_type=jax.ShapeDtypeStruct((B, EMBED_DIM), jnp.float32),
        scratch_types=[
            pltpu.VMEM((n_chunks, CHUNK), jnp.int32),
            pltpu.VMEM((2, GROUP, EMBED_DIM), jnp.float32),
            pltpu.SemaphoreType.DMA,
            pltpu.SemaphoreType.DMA,
            pltpu.SemaphoreType.DMA,
            pltpu.SemaphoreType.DMA,
        ],
        compiler_params=pltpu.CompilerParams(use_tc_tiling_on_sc=False),
    )
    def body(table_hbm, idx_hbm, out_hbm, idx_v, rows_v, g0, g1, o0, o1):
        wid = lax.axis_index("s") * NC + lax.axis_index("c")
        base_chunk = wid * n_chunks
        pltpu.sync_copy(idx_hbm.at[pl.ds(base_chunk, n_chunks)], idx_v)

        gsem = (g0, g1)
        osem = (o0, o1)

        def fire_g(g, b):
            for j in range(K):
                pltpu.async_copy(
                    table_hbm.at[idx_v.at[g * K + j]],
                    rows_v.at[b].at[pl.ds(j * CHUNK, CHUNK)],
                    gsem[b],
                )

        def drain_g(b):
            for j in range(K):
                pltpu.make_async_copy(
                    table_hbm.at[idx_v.at[0]],
                    rows_v.at[b].at[pl.ds(j * CHUNK, CHUNK)],
                    gsem[b],
                ).wait()

        def fire_out(g, b):
            pltpu.async_copy(
                rows_v.at[b],
                out_hbm.at[pl.ds((base_chunk + g * K) * CHUNK, GROUP)],
                osem[b],
            )

        def wait_out(b):
            pltpu.make_async_copy(
                out_hbm.at[pl.ds(0, GROUP)],
                rows_v.at[b],
                osem[b],
            ).wait()

        fire_g(0, 0)

        def step(i, first, last):
            if not first:
                wait_out(1)
            fire_g(2 * i + 1, 1)
            drain_g(0)
            fire_out(2 * i, 0)
            if not last:
                wait_out(0)
                fire_g(2 * i + 2, 0)
            drain_g(1)
            fire_out(2 * i + 1, 1)

        n_steps = n_groups // 2
        step(0, first=True, last=(n_steps == 1))

        def loop_body(i, carry):
            step(i, first=False, last=False)
            return carry

        if n_steps > 2:
            lax.fori_loop(1, n_steps - 1, loop_body, 0)
        if n_steps > 1:
            step(n_steps - 1, first=False, last=True)

        wait_out(0)
        wait_out(1)

    return body(table, idx2d)


def kernel(x, table):
    B, H = x.shape
    total = B * H
    info = plsc.get_sparse_core_info()
    NC, NS = info.num_cores, info.num_subcores
    NW = NC * NS
    n_chunks = total // (NW * CHUNK)
    n_groups = n_chunks // K
    idx_flat = x.reshape(total).astype(jnp.int32)
    idx2d = idx_flat.reshape(NW * n_chunks, CHUNK)
    out = _sc_gather(table, idx2d, n_chunks, n_groups, NC, NS)
    return out.reshape(B, H, EMBED_DIM)


# speedup vs baseline: 1.7970x; 1.6167x over previous
import functools

import jax
import jax.numpy as jnp
from jax import lax
from jax.experimental import pallas as pl
from jax.experimental.pallas import tpu as pltpu
from jax.experimental.pallas import tpu_sc as plsc

EMBED_DIM = 32
K = 8


def _sc_gather(table, idx, rows_per_w, n_groups, NC, NS):
    B, H = idx.shape
    mesh = plsc.VectorSubcoreMesh(core_axis_name="c", subcore_axis_name="s")

    @functools.partial(
        pl.kernel,
        mesh=mesh,
        out_type=jax.ShapeDtypeStruct((B, H, EMBED_DIM), jnp.float32),
        scratch_types=[
            pltpu.VMEM((rows_per_w, H), jnp.int32),
            pltpu.VMEM((2, K, H, EMBED_DIM), jnp.float32),
            pltpu.SemaphoreType.DMA,
            pltpu.SemaphoreType.DMA,
            pltpu.SemaphoreType.DMA,
            pltpu.SemaphoreType.DMA,
        ],
        compiler_params=pltpu.CompilerParams(use_tc_tiling_on_sc=False),
    )
    def body(table_hbm, idx_hbm, out_hbm, idx_v, rows_v, g0, g1, o0, o1):
        wid = lax.axis_index("s") * NC + lax.axis_index("c")
        base_row = wid * rows_per_w
        pltpu.sync_copy(idx_hbm.at[pl.ds(base_row, rows_per_w)], idx_v)

        gsem = (g0, g1)
        osem = (o0, o1)

        def fire_g(g, b):
            for j in range(K):
                pltpu.async_copy(
                    table_hbm.at[idx_v.at[g * K + j]],
                    rows_v.at[b].at[j],
                    gsem[b],
                )

        def drain_g(b):
            for j in range(K):
                pltpu.make_async_copy(
                    table_hbm.at[idx_v.at[0]],
                    rows_v.at[b].at[j],
                    gsem[b],
                ).wait()

        def fire_out(g, b):
            pltpu.async_copy(
                rows_v.at[b],
                out_hbm.at[pl.ds(base_row + g * K, K)],
                osem[b],
            )

        def wait_out(b):
            pltpu.make_async_copy(
                out_hbm.at[pl.ds(base_row, K)],
                rows_v.at[b],
                osem[b],
            ).wait()

        fire_g(0, 0)

        def step(i, first, last):
            if not first:
                wait_out(1)
            fire_g(2 * i + 1, 1)
            drain_g(0)
            fire_out(2 * i, 0)
            if not last:
                wait_out(0)
                fire_g(2 * i + 2, 0)
            drain_g(1)
            fire_out(2 * i + 1, 1)

        n_steps = n_groups // 2
        step(0, first=True, last=(n_steps == 1))

        def loop_body(i, carry):
            step(i, first=False, last=False)
            return carry

        if n_steps > 2:
            lax.fori_loop(1, n_steps - 1, loop_body, 0)
        if n_steps > 1:
            step(n_steps - 1, first=False, last=True)

        wait_out(0)
        wait_out(1)

    return body(table, idx)


def kernel(x, table):
    B, H = x.shape
    info = plsc.get_sparse_core_info()
    NC, NS = info.num_cores, info.num_subcores
    NW = NC * NS
    rows_per_w = B // NW
    n_groups = rows_per_w // K
    idx = x.astype(jnp.int32)
    return _sc_gather(table, idx, rows_per_w, n_groups, NC, NS)
